# baseline (speedup 1.0000x reference)
import functools

import jax
import jax.numpy as jnp
from jax import lax
from jax.experimental import pallas as pl
from jax.experimental.pallas import tpu as pltpu
from jax.experimental.pallas import tpu_sc as plsc

N = 10000
EMB = 64
B = 16
CHUNK = 128
NC, NS = 2, 16
NW = NC * NS
NP_ACC = 10112
RPT = NP_ACC // NS

_HIGH = lax.Precision.DEFAULT


K = 4
NB0 = 40
_WB = [(i * CHUNK, min(CHUNK, RPT - i * CHUNK)) for i in range(-(-RPT // CHUNK))]


@functools.lru_cache(maxsize=None)
def _make_agg(n_blocks_total: int):
    nb1 = n_blocks_total - NB0
    single = nb1 == 0
    out_shape = ((NP_ACC, EMB) if single else (NC, NP_ACC, EMB))
    mesh = plsc.VectorSubcoreMesh(core_axis_name="c", subcore_axis_name="s")

    @functools.partial(
        pl.kernel,
        out_type=jax.ShapeDtypeStruct(out_shape, jnp.float32),
        mesh=mesh,
        compiler_params=pltpu.CompilerParams(use_tc_tiling_on_sc=False),
        scratch_types=[
            pltpu.VMEM((2, K, CHUNK), jnp.int32),
            pltpu.VMEM((2, K, CHUNK), jnp.int32),
            pltpu.VMEM((K // 2, CHUNK, EMB), jnp.float32),
            pltpu.VMEM((K // 2, CHUNK, EMB), jnp.float32),
            pltpu.VMEM_SHARED((NP_ACC, EMB), jnp.float32),
            pltpu.SemaphoreType.DMA,
            pltpu.SemaphoreType.DMA,
            pltpu.SemaphoreType.DMA,
            pltpu.SemaphoreType.DMA,
            pltpu.SemaphoreType.DMA,
        ],
    )
    def agg(u_hbm, s_hbm, d_hbm, out_hbm,
            is2, id2, rA, rB, acc_sh, gsem, sA, sB, isem0, isem1):
        cid = lax.axis_index("c")
        sid = lax.axis_index("s")
        H = K // 2
        isems = (isem0, isem1)

        def work():
            zrow = jnp.zeros((16,), jnp.float32)

            @pl.loop(0, CHUNK)
            def _(r):
                for c in range(EMB // 16):
                    rA[0, r, pl.ds(c * 16, 16)] = zrow

            for off, sz in _WB:
                pltpu.sync_copy(rA.at[0, :sz],
                                acc_sh.at[pl.ds(sid * RPT + off, sz)])
            plsc.subcore_barrier()

            if single:
                nb_w = NB0
                base = sid * NB0
            else:
                nb_w = jnp.where(cid == 0, NB0, nb1)
                base = cid * (NS * NB0) + sid * nb_w

            def drain_scatters(rows, p, half, sem):
                for q in range(H):
                    pltpu.make_async_copy(
                        rows.at[q], acc_sh.at[id2.at[p, half * H + q]],
                        sem).wait()

            def drain_idx(p, b):
                pltpu.make_async_copy(s_hbm.at[b], is2.at[p],
                                      isems[p]).wait()
                pltpu.make_async_copy(d_hbm.at[b], id2.at[p],
                                      isems[p]).wait()

            def fire_gathers(rows, p, half):
                return [pltpu.async_copy(u_hbm.at[is2.at[p, half * H + q]],
                                         rows.at[q], gsem)
                        for q in range(H)]

            def fire_scatters(rows, p, half, sem):
                for q in range(H):
                    pltpu.async_copy(rows.at[q],
                                     acc_sh.at[id2.at[p, half * H + q]],
                                     sem, add=True)

            pltpu.sync_copy(s_hbm.at[base], is2.at[0])
            pltpu.sync_copy(d_hbm.at[base], id2.at[0])

            @pl.loop(0, nb_w // 2)
            def _(t):
                b0 = base + 2 * t
                for p in (0, 1):
                    b = b0 + p
                    if p == 0:
                        @pl.when(t > 0)
                        def _():
                            drain_scatters(rA, 1, 0, sA)
                            drain_idx(0, b)
                    else:
                        drain_scatters(rA, 0, 0, sA)
                        drain_idx(1, b)
                    gs = fire_gathers(rA, p, 0)
                    for g in gs:
                        g.wait()
                    fire_scatters(rA, p, 0, sA)
                    if p == 0:
                        @pl.when(t > 0)
                        def _():
                            drain_scatters(rB, 1, 1, sB)
                    else:
                        drain_scatters(rB, 0, 1, sB)
                    if p == 0:
                        pltpu.async_copy(s_hbm.at[b + 1], is2.at[1], isem1)
                        pltpu.async_copy(d_hbm.at[b + 1], id2.at[1], isem1)
                    else:
                        @pl.when(t + 1 < nb_w // 2)
                        def _():
                            pltpu.async_copy(s_hbm.at[b + 1], is2.at[0],
                                             isem0)
                            pltpu.async_copy(d_hbm.at[b + 1], id2.at[0],
                                             isem0)
                    gs = fire_gathers(rB, p, 1)
                    for g in gs:
                        g.wait()
                    fire_scatters(rB, p, 1, sB)

            drain_scatters(rA, 1, 0, sA)
            drain_scatters(rB, 1, 1, sB)

            plsc.subcore_barrier()
            if single:
                outs = [out_hbm.at[pl.ds(sid * RPT + off, sz)]
                        for off, sz in _WB]
            else:
                outs = [out_hbm.at[cid, pl.ds(sid * RPT + off, sz)]
                        for off, sz in _WB]
            handles, wp = [], 0
            for i, (off, sz) in enumerate(_WB):
                buf = (rA, rB)[(i // H) % 2]
                slot = i % H
                if i >= 2 * H:
                    handles[wp].wait()
                    wp += 1
                pltpu.sync_copy(acc_sh.at[pl.ds(sid * RPT + off, sz)],
                                buf.at[slot, :sz])
                handles.append(pltpu.async_copy(buf.at[slot, :sz],
                                                outs[i], gsem))
            for h in handles[wp:]:
                h.wait()

        if single:
            @pl.when(cid == 0)
            def _():
                work()
        else:
            work()

    return agg


def _first_matmul(x, w1):
    def body(x_ref, w_ref, o_ref):
        o_ref[...] = jnp.dot(x_ref[...], w_ref[...],
                             preferred_element_type=jnp.float32,
                             precision=_HIGH)
    return pl.pallas_call(
        body, out_shape=jax.ShapeDtypeStruct((N, EMB), jnp.float32))(x, w1)


def _layer_update(u, acc, eps11, b1, w2, b2, g, bb, w1n):
    has_next = w1n is not None

    def body(u_ref, a_ref, eps_ref, b1_ref, w2_ref, b2_ref, g_ref, bb_ref,
             *rest):
        if has_next:
            w1n_ref, o_ref = rest
        else:
            (o_ref,) = rest
        if a_ref.ndim == 3:
            agg = a_ref[0, :N, :] + a_ref[1, :N, :]
        else:
            agg = a_ref[:N, :]
        z = (1.0 + eps_ref[0, 0]) * u_ref[...] + agg + b1_ref[...]
        s = z * jax.nn.sigmoid(z)
        t = jnp.dot(s, w2_ref[...], preferred_element_type=jnp.float32,
                    precision=_HIGH) + b2_ref[...]
        mu = jnp.mean(t, axis=-1, keepdims=True)
        var = jnp.mean((t - mu) ** 2, axis=-1, keepdims=True)
        h = (t - mu) * lax.rsqrt(var + 1e-5) * g_ref[...] + bb_ref[...]
        if has_next:
            o_ref[...] = jnp.dot(h, w1n_ref[...],
                                 preferred_element_type=jnp.float32,
                                 precision=_HIGH)
        else:
            o_ref[...] = h

    args = (u, acc, eps11, b1, w2, b2, g, bb) + ((w1n,) if has_next else ())
    return pl.pallas_call(
        body, out_shape=jax.ShapeDtypeStruct((N, EMB), jnp.float32))(*args)


def _pool_head(h, batch_n1, batch_1n, gw1, gb1, gw2, gb2, fcw, fcb):
    def body(h_ref, bn1_ref, b1n_ref, gw1_ref, gb1_ref, gw2_ref, gb2_ref,
             fcw_ref, fcb_ref, o_ref):
        h = h_ref[...]
        hid = jnp.dot(h, gw1_ref[...], preferred_element_type=jnp.float32,
                      precision=_HIGH) + gb1_ref[...]
        hid = hid * jax.nn.sigmoid(hid)
        gate = jnp.dot(hid, gw2_ref[...], preferred_element_type=jnp.float32,
                       precision=_HIGH) + gb2_ref[...]
        oh_nk = (bn1_ref[...] ==
                 lax.broadcasted_iota(jnp.int32, (N, B), 1)).astype(jnp.float32)
        oh_kn = (b1n_ref[...] ==
                 lax.broadcasted_iota(jnp.int32, (B, N), 0)).astype(jnp.float32)
        masked = jnp.where(oh_nk > 0.5, gate, -1e30)
        m = jnp.max(masked, axis=0).reshape(B, 1)
        m_n = jnp.dot(oh_nk, m, preferred_element_type=jnp.float32,
                      precision=_HIGH)
        e = jnp.exp(gate - m_n)
        ssum = jnp.dot(oh_kn, e, preferred_element_type=jnp.float32,
                       precision=_HIGH)
        p = jnp.dot(oh_kn, e * h, preferred_element_type=jnp.float32,
                    precision=_HIGH)
        pooled = p / jnp.maximum(ssum, 1e-30)
        o_ref[...] = jnp.dot(pooled, fcw_ref[...],
                             preferred_element_type=jnp.float32,
                             precision=_HIGH) + fcb_ref[...]

    return pl.pallas_call(
        body, out_shape=jax.ShapeDtypeStruct((B, fcw.shape[1]), jnp.float32))(
            h, batch_n1, batch_1n, gw1, gb1, gw2, gb2, fcw, fcb)


def kernel(x, edge_index, batch,
           conv0_W1, conv0_b1, conv0_W2, conv0_b2, conv0_eps, ln0_g, ln0_b,
           conv1_W1, conv1_b1, conv1_W2, conv1_b2, conv1_eps, ln1_g, ln1_b,
           conv2_W1, conv2_b1, conv2_W2, conv2_b2, conv2_eps, ln2_g, ln2_b,
           gate_W1, gate_b1, gate_W2, gate_b2, fc_W, fc_b):
    e = edge_index.shape[1]
    blk = NS * CHUNK * K
    e_pad = -(-e // blk) * blk
    n_blocks_total = e_pad // blk
    tb = NS * n_blocks_total
    src = jnp.concatenate(
        [edge_index[0], jnp.zeros((e_pad - e,), jnp.int32)])
    dst = jnp.concatenate(
        [edge_index[1], jnp.full((e_pad - e,), N, jnp.int32)])
    src3 = src.reshape(tb, K, CHUNK)
    dst3 = dst.reshape(tb, K, CHUNK)
    agg_fn = _make_agg(n_blocks_total)

    params = [
        (conv0_eps, conv0_b1, conv0_W2, conv0_b2, ln0_g, ln0_b, conv1_W1),
        (conv1_eps, conv1_b1, conv1_W2, conv1_b2, ln1_g, ln1_b, conv2_W1),
        (conv2_eps, conv2_b1, conv2_W2, conv2_b2, ln2_g, ln2_b, None),
    ]

    u = _first_matmul(x, conv0_W1)
    for eps, b1, w2, b2, g, bb, w1n in params:
        acc = agg_fn(u, src3, dst3)
        u = _layer_update(u, acc,
                          eps.reshape(1, 1), b1.reshape(1, EMB), w2,
                          b2.reshape(1, EMB), g.reshape(1, EMB),
                          bb.reshape(1, EMB), w1n)

    h = u
    return _pool_head(h, batch.reshape(N, 1), batch.reshape(1, N),
                      gate_W1, gate_b1.reshape(1, EMB // 2),
                      gate_W2, gate_b2.reshape(1, 1),
                      fc_W, fc_b.reshape(1, fc_W.shape[1]))


# speedup vs baseline: 1.2758x; 1.2758x over previous
import functools

import jax
import jax.numpy as jnp
from jax import lax
from jax.experimental import pallas as pl
from jax.experimental.pallas import tpu as pltpu
from jax.experimental.pallas import tpu_sc as plsc

N = 10000
EMB = 64
B = 16
CHUNK = 128
NC, NS = 2, 16
NW = NC * NS
NP_ACC = 10112
RPT = NP_ACC // NS

_HIGH = lax.Precision.DEFAULT


K = 4
NB0 = 30
_WB = [(i * CHUNK, min(CHUNK, RPT - i * CHUNK)) for i in range(-(-RPT // CHUNK))]


@functools.lru_cache(maxsize=None)
def _make_agg(n_blocks_total: int):
    nb1 = n_blocks_total - NB0
    single = nb1 == 0
    out_shape = ((NP_ACC, EMB) if single else (NC, NP_ACC, EMB))
    mesh = plsc.VectorSubcoreMesh(core_axis_name="c", subcore_axis_name="s")

    @functools.partial(
        pl.kernel,
        out_type=jax.ShapeDtypeStruct(out_shape, jnp.float32),
        mesh=mesh,
        compiler_params=pltpu.CompilerParams(use_tc_tiling_on_sc=False),
        scratch_types=[
            pltpu.VMEM((2, K, CHUNK), jnp.int32),
            pltpu.VMEM((2, K, CHUNK), jnp.int32),
            pltpu.VMEM((K // 2, CHUNK, EMB), jnp.float32),
            pltpu.VMEM((K // 2, CHUNK, EMB), jnp.float32),
            pltpu.VMEM_SHARED((NP_ACC, EMB), jnp.float32),
            pltpu.SemaphoreType.DMA,
            pltpu.SemaphoreType.DMA,
            pltpu.SemaphoreType.DMA,
            pltpu.SemaphoreType.DMA,
            pltpu.SemaphoreType.DMA,
        ],
    )
    def agg(u_hbm, s_hbm, d_hbm, out_hbm,
            is2, id2, rA, rB, acc_sh, gsem, sA, sB, isem0, isem1):
        cid = lax.axis_index("c")
        sid = lax.axis_index("s")
        H = K // 2
        isems = (isem0, isem1)

        def work():
            zrow = jnp.zeros((16,), jnp.float32)

            @pl.loop(0, CHUNK)
            def _(r):
                for c in range(EMB // 16):
                    rA[0, r, pl.ds(c * 16, 16)] = zrow

            for off, sz in _WB:
                pltpu.sync_copy(rA.at[0, :sz],
                                acc_sh.at[pl.ds(sid * RPT + off, sz)])
            plsc.subcore_barrier()

            if single:
                nb_w = NB0
                base = sid * NB0
            else:
                nb_w = jnp.where(cid == 0, NB0, nb1)
                base = cid * (NS * NB0) + sid * nb_w

            def drain_scatters(rows, p, half, sem):
                for q in range(H):
                    pltpu.make_async_copy(
                        rows.at[q], acc_sh.at[id2.at[p, half * H + q]],
                        sem).wait()

            def drain_idx(p, b):
                pltpu.make_async_copy(s_hbm.at[b], is2.at[p],
                                      isems[p]).wait()
                pltpu.make_async_copy(d_hbm.at[b], id2.at[p],
                                      isems[p]).wait()

            def fire_gathers(rows, p, half):
                return [pltpu.async_copy(u_hbm.at[is2.at[p, half * H + q]],
                                         rows.at[q], gsem)
                        for q in range(H)]

            def fire_scatters(rows, p, half, sem):
                for q in range(H):
                    pltpu.async_copy(rows.at[q],
                                     acc_sh.at[id2.at[p, half * H + q]],
                                     sem, add=True)

            pltpu.sync_copy(s_hbm.at[base], is2.at[0])
            pltpu.sync_copy(d_hbm.at[base], id2.at[0])

            @pl.loop(0, nb_w // 2)
            def _(t):
                b0 = base + 2 * t
                for p in (0, 1):
                    b = b0 + p
                    if p == 0:
                        @pl.when(t > 0)
                        def _():
                            drain_scatters(rA, 1, 0, sA)
                            drain_idx(0, b)
                    else:
                        drain_scatters(rA, 0, 0, sA)
                        drain_idx(1, b)
                    gs = fire_gathers(rA, p, 0)
                    for g in gs:
                        g.wait()
                    fire_scatters(rA, p, 0, sA)
                    if p == 0:
                        @pl.when(t > 0)
                        def _():
                            drain_scatters(rB, 1, 1, sB)
                    else:
                        drain_scatters(rB, 0, 1, sB)
                    if p == 0:
                        pltpu.async_copy(s_hbm.at[b + 1], is2.at[1], isem1)
                        pltpu.async_copy(d_hbm.at[b + 1], id2.at[1], isem1)
                    else:
                        @pl.when(t + 1 < nb_w // 2)
                        def _():
                            pltpu.async_copy(s_hbm.at[b + 1], is2.at[0],
                                             isem0)
                            pltpu.async_copy(d_hbm.at[b + 1], id2.at[0],
                                             isem0)
                    gs = fire_gathers(rB, p, 1)
                    for g in gs:
                        g.wait()
                    fire_scatters(rB, p, 1, sB)

            drain_scatters(rA, 1, 0, sA)
            drain_scatters(rB, 1, 1, sB)

            plsc.subcore_barrier()
            if single:
                outs = [out_hbm.at[pl.ds(sid * RPT + off, sz)]
                        for off, sz in _WB]
            else:
                outs = [out_hbm.at[cid, pl.ds(sid * RPT + off, sz)]
                        for off, sz in _WB]
            handles, wp = [], 0
            for i, (off, sz) in enumerate(_WB):
                buf = (rA, rB)[(i // H) % 2]
                slot = i % H
                if i >= 2 * H:
                    handles[wp].wait()
                    wp += 1
                pltpu.sync_copy(acc_sh.at[pl.ds(sid * RPT + off, sz)],
                                buf.at[slot, :sz])
                handles.append(pltpu.async_copy(buf.at[slot, :sz],
                                                outs[i], gsem))
            for h in handles[wp:]:
                h.wait()

        if single:
            @pl.when(cid == 0)
            def _():
                work()
        else:
            work()

    return agg


def _first_matmul(x, w1):
    def body(x_ref, w_ref, o_ref):
        o_ref[...] = jnp.dot(x_ref[...], w_ref[...],
                             preferred_element_type=jnp.float32,
                             precision=_HIGH)
    return pl.pallas_call(
        body, out_shape=jax.ShapeDtypeStruct((N, EMB), jnp.float32))(x, w1)


def _layer_update(u, acc, eps11, b1, w2, b2, g, bb, w1n):
    has_next = w1n is not None

    def body(u_ref, a_ref, eps_ref, b1_ref, w2_ref, b2_ref, g_ref, bb_ref,
             *rest):
        if has_next:
            w1n_ref, o_ref = rest
        else:
            (o_ref,) = rest
        if a_ref.ndim == 3:
            agg = a_ref[0, :N, :] + a_ref[1, :N, :]
        else:
            agg = a_ref[:N, :]
        z = (1.0 + eps_ref[0, 0]) * u_ref[...] + agg + b1_ref[...]
        s = z * jax.nn.sigmoid(z)
        t = jnp.dot(s, w2_ref[...], preferred_element_type=jnp.float32,
                    precision=_HIGH) + b2_ref[...]
        mu = jnp.mean(t, axis=-1, keepdims=True)
        var = jnp.mean((t - mu) ** 2, axis=-1, keepdims=True)
        h = (t - mu) * lax.rsqrt(var + 1e-5) * g_ref[...] + bb_ref[...]
        if has_next:
            o_ref[...] = jnp.dot(h, w1n_ref[...],
                                 preferred_element_type=jnp.float32,
                                 precision=_HIGH)
        else:
            o_ref[...] = h

    args = (u, acc, eps11, b1, w2, b2, g, bb) + ((w1n,) if has_next else ())
    return pl.pallas_call(
        body, out_shape=jax.ShapeDtypeStruct((N, EMB), jnp.float32))(*args)


def _pool_head(h, batch_n1, batch_1n, gw1, gb1, gw2, gb2, fcw, fcb):
    def body(h_ref, bn1_ref, b1n_ref, gw1_ref, gb1_ref, gw2_ref, gb2_ref,
             fcw_ref, fcb_ref, o_ref):
        h = h_ref[...]
        hid = jnp.dot(h, gw1_ref[...], preferred_element_type=jnp.float32,
                      precision=_HIGH) + gb1_ref[...]
        hid = hid * jax.nn.sigmoid(hid)
        gate = jnp.dot(hid, gw2_ref[...], preferred_element_type=jnp.float32,
                       precision=_HIGH) + gb2_ref[...]
        oh_nk = (bn1_ref[...] ==
                 lax.broadcasted_iota(jnp.int32, (N, B), 1)).astype(jnp.float32)
        oh_kn = (b1n_ref[...] ==
                 lax.broadcasted_iota(jnp.int32, (B, N), 0)).astype(jnp.float32)
        masked = jnp.where(oh_nk > 0.5, gate, -1e30)
        m = jnp.max(masked, axis=0).reshape(B, 1)
        m_n = jnp.dot(oh_nk, m, preferred_element_type=jnp.float32,
                      precision=_HIGH)
        e = jnp.exp(gate - m_n)
        ssum = jnp.dot(oh_kn, e, preferred_element_type=jnp.float32,
                       precision=_HIGH)
        p = jnp.dot(oh_kn, e * h, preferred_element_type=jnp.float32,
                    precision=_HIGH)
        pooled = p / jnp.maximum(ssum, 1e-30)
        o_ref[...] = jnp.dot(pooled, fcw_ref[...],
                             preferred_element_type=jnp.float32,
                             precision=_HIGH) + fcb_ref[...]

    return pl.pallas_call(
        body, out_shape=jax.ShapeDtypeStruct((B, fcw.shape[1]), jnp.float32))(
            h, batch_n1, batch_1n, gw1, gb1, gw2, gb2, fcw, fcb)


def kernel(x, edge_index, batch,
           conv0_W1, conv0_b1, conv0_W2, conv0_b2, conv0_eps, ln0_g, ln0_b,
           conv1_W1, conv1_b1, conv1_W2, conv1_b2, conv1_eps, ln1_g, ln1_b,
           conv2_W1, conv2_b1, conv2_W2, conv2_b2, conv2_eps, ln2_g, ln2_b,
           gate_W1, gate_b1, gate_W2, gate_b2, fc_W, fc_b):
    e = edge_index.shape[1]
    blk = NS * CHUNK * K
    e_pad = -(-e // blk) * blk
    n_blocks_total = e_pad // blk
    tb = NS * n_blocks_total
    src = jnp.concatenate(
        [edge_index[0], jnp.zeros((e_pad - e,), jnp.int32)])
    dst = jnp.concatenate(
        [edge_index[1], jnp.full((e_pad - e,), N, jnp.int32)])
    src3 = src.reshape(tb, K, CHUNK)
    dst3 = dst.reshape(tb, K, CHUNK)
    agg_fn = _make_agg(n_blocks_total)

    params = [
        (conv0_eps, conv0_b1, conv0_W2, conv0_b2, ln0_g, ln0_b, conv1_W1),
        (conv1_eps, conv1_b1, conv1_W2, conv1_b2, ln1_g, ln1_b, conv2_W1),
        (conv2_eps, conv2_b1, conv2_W2, conv2_b2, ln2_g, ln2_b, None),
    ]

    u = _first_matmul(x, conv0_W1)
    for eps, b1, w2, b2, g, bb, w1n in params:
        acc = agg_fn(u, src3, dst3)
        u = _layer_update(u, acc,
                          eps.reshape(1, 1), b1.reshape(1, EMB), w2,
                          b2.reshape(1, EMB), g.reshape(1, EMB),
                          bb.reshape(1, EMB), w1n)

    h = u
    return _pool_head(h, batch.reshape(N, 1), batch.reshape(1, N),
                      gate_W1, gate_b1.reshape(1, EMB // 2),
                      gate_W2, gate_b2.reshape(1, 1),
                      fc_W, fc_b.reshape(1, fc_W.shape[1]))
